# baseline (speedup 1.0000x reference)
import functools
import jax
import jax.numpy as jnp
from jax import lax
from jax.experimental import pallas as pl
from jax.experimental.pallas import tpu as pltpu, tpu_sc as plsc

NC = 2
NS = 16
NW = NC * NS

NODES = 1000000
B = 4096
L = 50
H = 32
TOTAL = B * L
B_PER_W = TOTAL // NW
CHUNK = 128
NCHUNK = B_PER_W // CHUNK

NB = 8
LOOKAHEAD = 4

NBLK = NODES // 128
TAIL = NODES - NBLK * 128
BPT = -(-NBLK // NW)

_mesh = plsc.VectorSubcoreMesh(core_axis_name="c", subcore_axis_name="s")


@functools.partial(
    pl.kernel,
    out_type=jax.ShapeDtypeStruct((NODES * H,), jnp.float32),
    mesh=_mesh,
    scratch_types=[
        pltpu.VMEM((H, 128), jnp.float32),
        pltpu.VMEM((128 * H,), jnp.float32),
    ],
    compiler_params=pltpu.CompilerParams(
        use_tc_tiling_on_sc=True, needs_layout_passes=False
    ),
)
def _transpose_kernel(tab_t_hbm, tail_hbm, flat_hbm, stage_v, tbuf_v):
    wid = lax.axis_index("s") * NC + lax.axis_index("c")
    start = wid * BPT
    stop = jnp.minimum(start + BPT, NBLK)

    iota0 = jax.lax.iota(jnp.int32, 16)
    iota1 = iota0 + 16

    def blk_body(nb, carry):
        pltpu.sync_copy(tab_t_hbm.at[:, pl.ds(nb * 128, 128)], stage_v)
        for j in range(128):
            col = jnp.full((16,), j, jnp.int32)
            v0 = plsc.load_gather(stage_v, [iota0, col])
            v1 = plsc.load_gather(stage_v, [iota1, col])
            tbuf_v[pl.ds(j * H, 16)] = v0
            tbuf_v[pl.ds(j * H + 16, 16)] = v1
        pltpu.sync_copy(tbuf_v, flat_hbm.at[pl.ds(nb * 128 * H, 128 * H)])
        return carry

    lax.fori_loop(start, stop, blk_body, 0)

    @pl.when(wid == NW - 1)
    def _():
        pltpu.sync_copy(tail_hbm, tbuf_v.at[pl.ds(0, TAIL * H)])
        pltpu.sync_copy(tbuf_v.at[pl.ds(0, TAIL * H)],
                        flat_hbm.at[pl.ds(NBLK * 128 * H, TAIL * H)])


@functools.partial(
    pl.kernel,
    out_type=jax.ShapeDtypeStruct((TOTAL, H), jnp.float32),
    mesh=_mesh,
    scratch_types=[
        pltpu.VMEM((NCHUNK, CHUNK), jnp.int32),
        pltpu.VMEM((NB, CHUNK, H), jnp.float32),
        pltpu.SemaphoreType.DMA((NB,)),
        pltpu.SemaphoreType.DMA((NB,)),
    ],
    compiler_params=pltpu.CompilerParams(use_tc_tiling_on_sc=False),
)
def _gather_kernel(idx_hbm, table_hbm, out_hbm, idx_v, rows_v, gsems, osems):
    wid = lax.axis_index("s") * NC + lax.axis_index("c")
    base = wid * B_PER_W
    pltpu.sync_copy(idx_hbm.at[wid], idx_v)

    def gather_start(j, b):
        pltpu.async_copy(table_hbm.at[idx_v.at[j]], rows_v.at[b], gsems.at[b])

    def gather_wait(j, b):
        pltpu.make_async_copy(
            table_hbm.at[idx_v.at[j]], rows_v.at[b], gsems.at[b]
        ).wait()

    def out_start(j, b):
        pltpu.async_copy(
            rows_v.at[b], out_hbm.at[pl.ds(base + j * CHUNK, CHUNK)], osems.at[b]
        )

    def out_wait(j, b):
        pltpu.make_async_copy(
            rows_v.at[b], out_hbm.at[pl.ds(base + j * CHUNK, CHUNK)], osems.at[b]
        ).wait()

    for b in range(LOOKAHEAD):
        gather_start(b, b)

    def body(j, carry):
        b = j % NB
        gather_wait(j, b)
        out_start(j, b)
        jn = j + LOOKAHEAD
        bn = jn % NB

        @pl.when(jn < NCHUNK)
        def _():
            @pl.when(jn >= NB)
            def _():
                out_wait(jn - NB, bn)

            gather_start(jn, bn)

        return carry

    lax.fori_loop(0, NCHUNK, body, 0)

    for t in range(NCHUNK - NB, NCHUNK):
        out_wait(t, t % NB)


def kernel(conditions, table):
    tail = table[NBLK * 128:].reshape(TAIL * H)
    flat = _transpose_kernel(table.T, tail)
    t_lin = flat.reshape(NODES, H)
    idx = conditions.reshape(NW, NCHUNK, CHUNK)
    out = _gather_kernel(idx, t_lin)
    return out.reshape(B, L * H)


# speedup vs baseline: 1.4283x; 1.4283x over previous
import functools
import jax
import jax.numpy as jnp
from jax import lax
from jax.experimental import pallas as pl
from jax.experimental.pallas import tpu as pltpu, tpu_sc as plsc

NC = 2
NS = 16
NW = NC * NS

NODES = 1000000
B = 4096
L = 50
H = 32
TOTAL = B * L
B_PER_W = TOTAL // NW
CHUNK = 128
NCHUNK = B_PER_W // CHUNK

NB = 8
LOOKAHEAD = 4

W = 512
NBLK = NODES // W
TAIL = NODES - NBLK * W
BPT = -(-NBLK // NW)

_mesh = plsc.VectorSubcoreMesh(core_axis_name="c", subcore_axis_name="s")


@functools.partial(
    pl.kernel,
    out_type=jax.ShapeDtypeStruct((NODES * H,), jnp.float32),
    mesh=_mesh,
    scratch_types=[
        pltpu.VMEM((2 * H, W), jnp.float32),
        pltpu.VMEM((2 * W * H,), jnp.float32),
        pltpu.SemaphoreType.DMA((2,)),
        pltpu.SemaphoreType.DMA((2,)),
    ],
    compiler_params=pltpu.CompilerParams(
        use_tc_tiling_on_sc=True, needs_layout_passes=False
    ),
)
def _transpose_kernel(tab_t_hbm, tail_hbm, flat_hbm, stage_v, tbuf_v,
                      isems, osems):
    wid = lax.axis_index("s") * NC + lax.axis_index("c")
    start = wid * BPT
    stop = jnp.minimum(start + BPT, NBLK)

    iota_h = jax.lax.iota(jnp.int32, 16) * H

    def in_start(blk, b):
        pltpu.async_copy(tab_t_hbm.at[:, pl.ds(blk * W, W)],
                         stage_v.at[pl.ds(b * H, H)], isems.at[b])

    def in_wait(blk, b):
        pltpu.make_async_copy(tab_t_hbm.at[:, pl.ds(blk * W, W)],
                              stage_v.at[pl.ds(b * H, H)], isems.at[b]).wait()

    def out_start(blk, b):
        pltpu.async_copy(tbuf_v.at[pl.ds(b * W * H, W * H)],
                         flat_hbm.at[pl.ds(blk * W * H, W * H)], osems.at[b])

    def out_wait(blk, b):
        pltpu.make_async_copy(tbuf_v.at[pl.ds(b * W * H, W * H)],
                              flat_hbm.at[pl.ds(blk * W * H, W * H)],
                              osems.at[b]).wait()

    def transpose_block(b):
        def gbody(g, carry):
            base = b * W * H + g * 16 * H
            for h in range(H):
                v = stage_v[b * H + h, pl.ds(g * 16, 16)]
                plsc.store_scatter(tbuf_v, [iota_h + (base + h)], v)
            return carry

        lax.fori_loop(0, W // 16, gbody, 0)

    in_start(start, 0)

    def body(i, carry):
        @pl.when(i % 2 == 0)
        def _():
            run(i, 0)

        @pl.when(i % 2 == 1)
        def _():
            run(i, 1)

        return carry

    def run(i, b):
        in_wait(i, b)

        @pl.when(i + 1 < stop)
        def _():
            in_start(i + 1, 1 - b)

        @pl.when(i >= start + 2)
        def _():
            out_wait(i - 2, b)

        transpose_block(b)
        out_start(i, b)

    lax.fori_loop(start, stop, body, 0)

    @pl.when(stop - start >= 2)
    def _():
        out_wait(stop - 2, (stop - 2) % 2)

    @pl.when(stop - start >= 1)
    def _():
        out_wait(stop - 1, (stop - 1) % 2)

    @pl.when(wid == NW - 1)
    def _():
        pltpu.sync_copy(tail_hbm, tbuf_v.at[pl.ds(0, TAIL * H)])
        pltpu.sync_copy(tbuf_v.at[pl.ds(0, TAIL * H)],
                        flat_hbm.at[pl.ds(NBLK * W * H, TAIL * H)])


@functools.partial(
    pl.kernel,
    out_type=jax.ShapeDtypeStruct((TOTAL, H), jnp.float32),
    mesh=_mesh,
    scratch_types=[
        pltpu.VMEM((NCHUNK, CHUNK), jnp.int32),
        pltpu.VMEM((NB, CHUNK, H), jnp.float32),
        pltpu.SemaphoreType.DMA((NB,)),
        pltpu.SemaphoreType.DMA((NB,)),
    ],
    compiler_params=pltpu.CompilerParams(use_tc_tiling_on_sc=False),
)
def _gather_kernel(idx_hbm, table_hbm, out_hbm, idx_v, rows_v, gsems, osems):
    wid = lax.axis_index("s") * NC + lax.axis_index("c")
    base = wid * B_PER_W
    pltpu.sync_copy(idx_hbm.at[wid], idx_v)

    def gather_start(j, b):
        pltpu.async_copy(table_hbm.at[idx_v.at[j]], rows_v.at[b], gsems.at[b])

    def gather_wait(j, b):
        pltpu.make_async_copy(
            table_hbm.at[idx_v.at[j]], rows_v.at[b], gsems.at[b]
        ).wait()

    def out_start(j, b):
        pltpu.async_copy(
            rows_v.at[b], out_hbm.at[pl.ds(base + j * CHUNK, CHUNK)], osems.at[b]
        )

    def out_wait(j, b):
        pltpu.make_async_copy(
            rows_v.at[b], out_hbm.at[pl.ds(base + j * CHUNK, CHUNK)], osems.at[b]
        ).wait()

    for b in range(LOOKAHEAD):
        gather_start(b, b)

    def body(j, carry):
        b = j % NB
        gather_wait(j, b)
        out_start(j, b)
        jn = j + LOOKAHEAD
        bn = jn % NB

        @pl.when(jn < NCHUNK)
        def _():
            @pl.when(jn >= NB)
            def _():
                out_wait(jn - NB, bn)

            gather_start(jn, bn)

        return carry

    lax.fori_loop(0, NCHUNK, body, 0)

    for t in range(NCHUNK - NB, NCHUNK):
        out_wait(t, t % NB)


def kernel(conditions, table):
    tail = table[NBLK * W:].reshape(TAIL * H)
    flat = _transpose_kernel(table.T, tail)
    t_lin = flat.reshape(NODES, H)
    idx = conditions.reshape(NW, NCHUNK, CHUNK)
    out = _gather_kernel(idx, t_lin)
    return out.reshape(B, L * H)


# speedup vs baseline: 4.9676x; 3.4779x over previous
import functools
import jax
import jax.numpy as jnp
from jax import lax
from jax.experimental import pallas as pl
from jax.experimental.pallas import tpu as pltpu, tpu_sc as plsc

NC = 2
NS = 16
NW = NC * NS

NODES = 1000000
B = 4096
L = 50
H = 32
TOTAL = B * L
B_PER_W = TOTAL // NW
CHUNK = 128
NCHUNK = B_PER_W // CHUNK

NB = 8
LOOKAHEAD = 4

W = 512
NBLK = NODES // W
TAIL = NODES - NBLK * W
BPT = -(-NBLK // NW)

_mesh = plsc.VectorSubcoreMesh(core_axis_name="c", subcore_axis_name="s")


@functools.partial(
    pl.kernel,
    out_type=jax.ShapeDtypeStruct((NODES * H,), jnp.float32),
    mesh=_mesh,
    scratch_types=[
        pltpu.VMEM((2 * H, W), jnp.float32),
        pltpu.VMEM((2 * W * H,), jnp.float32),
        pltpu.SemaphoreType.DMA((2,)),
        pltpu.SemaphoreType.DMA((2,)),
    ],
    compiler_params=pltpu.CompilerParams(
        use_tc_tiling_on_sc=True, needs_layout_passes=False
    ),
)
def _transpose_kernel(tab_t_hbm, tail_hbm, flat_hbm, stage_v, tbuf_v,
                      isems, osems):
    wid = lax.axis_index("s") * NC + lax.axis_index("c")
    start = wid * BPT
    stop = jnp.minimum(start + BPT, NBLK)

    iota_h = jax.lax.iota(jnp.int32, 16) * H

    def in_start(blk, b):
        pltpu.async_copy(tab_t_hbm.at[:, pl.ds(blk * W, W)],
                         stage_v.at[pl.ds(b * H, H)], isems.at[b])

    def in_wait(blk, b):
        pltpu.make_async_copy(tab_t_hbm.at[:, pl.ds(blk * W, W)],
                              stage_v.at[pl.ds(b * H, H)], isems.at[b]).wait()

    def out_start(blk, b):
        pltpu.async_copy(tbuf_v.at[pl.ds(b * W * H, W * H)],
                         flat_hbm.at[pl.ds(blk * W * H, W * H)], osems.at[b])

    def out_wait(blk, b):
        pltpu.make_async_copy(tbuf_v.at[pl.ds(b * W * H, W * H)],
                              flat_hbm.at[pl.ds(blk * W * H, W * H)],
                              osems.at[b]).wait()

    def transpose_block(b):
        @functools.partial(plsc.parallel_loop, 0, W // 16, unroll=4)
        def _(g):
            base = b * W * H + g * 16 * H
            for h in range(H):
                v = stage_v[b * H + h, pl.ds(g * 16, 16)]
                plsc.store_scatter(tbuf_v, [iota_h + (base + h)], v)

    in_start(start, 0)

    def body(i, carry):
        @pl.when(i % 2 == 0)
        def _():
            run(i, 0)

        @pl.when(i % 2 == 1)
        def _():
            run(i, 1)

        return carry

    def run(i, b):
        in_wait(i, b)

        @pl.when(i + 1 < stop)
        def _():
            in_start(i + 1, 1 - b)

        @pl.when(i >= start + 2)
        def _():
            out_wait(i - 2, b)

        transpose_block(b)
        out_start(i, b)

    lax.fori_loop(start, stop, body, 0)

    @pl.when(stop - start >= 2)
    def _():
        out_wait(stop - 2, (stop - 2) % 2)

    @pl.when(stop - start >= 1)
    def _():
        out_wait(stop - 1, (stop - 1) % 2)

    @pl.when(wid == NW - 1)
    def _():
        pltpu.sync_copy(tail_hbm, tbuf_v.at[pl.ds(0, TAIL * H)])
        pltpu.sync_copy(tbuf_v.at[pl.ds(0, TAIL * H)],
                        flat_hbm.at[pl.ds(NBLK * W * H, TAIL * H)])


@functools.partial(
    pl.kernel,
    out_type=jax.ShapeDtypeStruct((TOTAL, H), jnp.float32),
    mesh=_mesh,
    scratch_types=[
        pltpu.VMEM((NCHUNK, CHUNK), jnp.int32),
        pltpu.VMEM((NB, CHUNK, H), jnp.float32),
        pltpu.SemaphoreType.DMA((NB,)),
        pltpu.SemaphoreType.DMA((NB,)),
    ],
    compiler_params=pltpu.CompilerParams(use_tc_tiling_on_sc=False),
)
def _gather_kernel(idx_hbm, table_hbm, out_hbm, idx_v, rows_v, gsems, osems):
    wid = lax.axis_index("s") * NC + lax.axis_index("c")
    base = wid * B_PER_W
    pltpu.sync_copy(idx_hbm.at[wid], idx_v)

    def gather_start(j, b):
        pltpu.async_copy(table_hbm.at[idx_v.at[j]], rows_v.at[b], gsems.at[b])

    def gather_wait(j, b):
        pltpu.make_async_copy(
            table_hbm.at[idx_v.at[j]], rows_v.at[b], gsems.at[b]
        ).wait()

    def out_start(j, b):
        pltpu.async_copy(
            rows_v.at[b], out_hbm.at[pl.ds(base + j * CHUNK, CHUNK)], osems.at[b]
        )

    def out_wait(j, b):
        pltpu.make_async_copy(
            rows_v.at[b], out_hbm.at[pl.ds(base + j * CHUNK, CHUNK)], osems.at[b]
        ).wait()

    for b in range(LOOKAHEAD):
        gather_start(b, b)

    def body(j, carry):
        b = j % NB
        gather_wait(j, b)
        out_start(j, b)
        jn = j + LOOKAHEAD
        bn = jn % NB

        @pl.when(jn < NCHUNK)
        def _():
            @pl.when(jn >= NB)
            def _():
                out_wait(jn - NB, bn)

            gather_start(jn, bn)

        return carry

    lax.fori_loop(0, NCHUNK, body, 0)

    for t in range(NCHUNK - NB, NCHUNK):
        out_wait(t, t % NB)


def kernel(conditions, table):
    tail = table[NBLK * W:].reshape(TAIL * H)
    flat = _transpose_kernel(table.T, tail)
    t_lin = flat.reshape(NODES, H)
    idx = conditions.reshape(NW, NCHUNK, CHUNK)
    out = _gather_kernel(idx, t_lin)
    return out.reshape(B, L * H)
